# Initial kernel scaffold; baseline (speedup 1.0000x reference)
import jax
import jax.numpy as jnp
from jax.experimental import pallas as pl


def kernel(u, i, last_basket, V_IL, V_LI):
    raise NotImplementedError("write your pallas kernel here")


# speedup vs baseline: 1.0770x; 1.0770x over previous
import functools

import jax
import jax.numpy as jnp
from jax import lax
from jax.experimental import pallas as pl
from jax.experimental.pallas import tpu as pltpu
from jax.experimental.pallas import tpu_sc as plsc

B = 16384
L = 20
D = 32
NW = 32
BPW = B // NW
G = 128
CB = 64
NCH = BPW // CB
CROWS = CB * L
CG = CROWS // G
IROWS_W = BPW * L // G
IROWS_I = BPW // G


def _body(i_hbm, lb_hbm, vil, vli, out_hbm,
          i_v, lb_v, ei_v, el_a, el_b, tbuf, out_v,
          sem_ei, sem_a, sem_b):
  w = lax.axis_index("s") * 2 + lax.axis_index("c")
  base = w * BPW

  pltpu.sync_copy(i_hbm.at[pl.ds(w * IROWS_I, IROWS_I)], i_v)
  pltpu.sync_copy(lb_hbm.at[pl.ds(w * IROWS_W, IROWS_W)], lb_v)

  ei_copies = [
      pltpu.async_copy(vil.at[i_v.at[j]], ei_v.at[pl.ds(j * G, G)], sem_ei)
      for j in range(IROWS_I)
  ]

  el_bufs = (el_a, el_b)
  sems = (sem_a, sem_b)

  def start_chunk(c):
    p = c % 2
    return [
        pltpu.async_copy(vli.at[lb_v.at[c * CG + j]],
                         el_bufs[p].at[pl.ds(j * G, G)], sems[p])
        for j in range(CG)
    ]

  pending = {0: start_chunk(0)}
  for cp in ei_copies:
    cp.wait()

  for c in range(NCH):
    if c + 1 < NCH:
      pending[c + 1] = start_chunk(c + 1)
    for cp in pending.pop(c):
      cp.wait()
    el = el_bufs[c % 2]

    def bbody(b, carry, el=el, c=c):
      r0 = b * L
      s0 = el[r0, pl.ds(0, 16)]
      s1 = el[r0, pl.ds(16, 16)]
      for l in range(1, L):
        s0 = s0 + el[r0 + l, pl.ds(0, 16)]
        s1 = s1 + el[r0 + l, pl.ds(16, 16)]
      cb = c * CB + b
      t = ei_v[cb, pl.ds(0, 16)] * s0 + ei_v[cb, pl.ds(16, 16)] * s1
      tbuf[b, :] = t
      return carry

    lax.fori_loop(0, CB, bbody, 0)

    lane = jnp.arange(16, dtype=jnp.int32)
    for bg in range(CB // 16):
      rows = lane + (bg * 16)
      acc = plsc.load_gather(tbuf, [rows, jnp.full((16,), 0, jnp.int32)])
      for k in range(1, 16):
        acc = acc + plsc.load_gather(tbuf, [rows, jnp.full((16,), k, jnp.int32)])
      out_v[pl.ds(c * CB + bg * 16, 16)] = acc * jnp.float32(1.0 / L)

  pltpu.sync_copy(out_v, out_hbm.at[pl.ds(base, BPW)])


@functools.partial(jax.jit, static_argnums=())
def _fpmc(i2, lb2, vil, vli):
  mesh = plsc.VectorSubcoreMesh(core_axis_name="c", subcore_axis_name="s")
  return pl.kernel(
      _body,
      out_type=jax.ShapeDtypeStruct((B,), jnp.float32),
      mesh=mesh,
      compiler_params=pltpu.CompilerParams(
          needs_layout_passes=False, use_tc_tiling_on_sc=False),
      scratch_types=[
          pltpu.VMEM((IROWS_I, G), jnp.int32),
          pltpu.VMEM((IROWS_W, G), jnp.int32),
          pltpu.VMEM((BPW, D), jnp.float32),
          pltpu.VMEM((CROWS, D), jnp.float32),
          pltpu.VMEM((CROWS, D), jnp.float32),
          pltpu.VMEM((CB, 16), jnp.float32),
          pltpu.VMEM((BPW,), jnp.float32),
          pltpu.SemaphoreType.DMA,
          pltpu.SemaphoreType.DMA,
          pltpu.SemaphoreType.DMA,
      ],
  )(i2, lb2, vil, vli)


def kernel(u, i, last_basket, V_IL, V_LI):
  del u
  i2 = i.astype(jnp.int32).reshape(B // G, G)
  lb2 = last_basket.astype(jnp.int32).reshape(B * L // G, G)
  return _fpmc(i2, lb2, V_IL, V_LI)
